# baseline (speedup 1.0000x reference)
import functools

import jax
import jax.numpy as jnp
from jax import lax
from jax.experimental import pallas as pl
from jax.experimental.pallas import tpu as pltpu
from jax.experimental.pallas import tpu_sc as plsc

NUM_CATEGORIES = 1000000
EMBED_DIM = 64
BATCH = 4096
HIST_LEN = 50

NC = 2
NS = 16
NW = NC * NS

B_TOTAL = BATCH * HIST_LEN
B_PER_W = B_TOTAL // NW
SEG = 128
NSEG = B_PER_W // SEG
K = 5
NSTEP = NSEG // K


def _body(idx_hbm, table_hbm, out_hbm, idx_v, rows_v, sem):
    wid = lax.axis_index("s") * NC + lax.axis_index("c")
    pltpu.sync_copy(idx_hbm.at[wid], idx_v)

    def step(g, _):
        cps = []
        for j in range(K):
            cps.append(pltpu.async_copy(
                table_hbm.at[idx_v.at[g * K + j]],
                rows_v.at[pl.ds(j * SEG, SEG)],
                sem,
            ))
        for cp in cps:
            cp.wait()
        pltpu.sync_copy(
            rows_v, out_hbm.at[pl.ds(wid * B_PER_W + g * (K * SEG), K * SEG)])
        return ()

    lax.fori_loop(0, NSTEP, step, (), unroll=False)


@jax.jit
def _gather(idx3, table):
    mesh = plsc.VectorSubcoreMesh(
        core_axis_name="c", subcore_axis_name="s", num_cores=NC,
        num_subcores=NS)
    return pl.kernel(
        _body,
        out_type=jax.ShapeDtypeStruct((B_TOTAL, EMBED_DIM), jnp.float32),
        mesh=mesh,
        scratch_types=[
            pltpu.VMEM((NSEG, SEG), jnp.int32),
            pltpu.VMEM((K * SEG, EMBED_DIM), jnp.float32),
            pltpu.SemaphoreType.DMA,
        ],
        compiler_params=pltpu.CompilerParams(use_tc_tiling_on_sc=False),
    )(idx3, table)


def kernel(inputs, table):
    idx3 = inputs.reshape(NW, NSEG, SEG)
    return _gather(idx3, table)


# speedup vs baseline: 1.0019x; 1.0019x over previous
import functools

import jax
import jax.numpy as jnp
from jax import lax
from jax.experimental import pallas as pl
from jax.experimental.pallas import tpu as pltpu
from jax.experimental.pallas import tpu_sc as plsc

NUM_CATEGORIES = 1000000
EMBED_DIM = 64
BATCH = 4096
HIST_LEN = 50

NC = 2
NS = 16
NW = NC * NS

B_TOTAL = BATCH * HIST_LEN
B_PER_W = B_TOTAL // NW
SEG = 128
NSEG = B_PER_W // SEG
K = 5
NSTEP = NSEG // K


CHUNK = K * SEG


def _body(idx_hbm, table_hbm, out_hbm, idx_v, rows_v, sem_g, sem_w):
    wid = lax.axis_index("s") * NC + lax.axis_index("c")
    pltpu.sync_copy(idx_hbm.at[wid], idx_v)

    def issue_gathers(g, b):
        for j in range(K):
            pltpu.async_copy(
                table_hbm.at[idx_v.at[g * K + j]],
                rows_v.at[b].at[pl.ds(j * SEG, SEG)],
                sem_g,
            )

    def wait_gathers(g, b):
        for j in range(K):
            pltpu.make_async_copy(
                table_hbm.at[idx_v.at[g * K + j]],
                rows_v.at[b].at[pl.ds(j * SEG, SEG)],
                sem_g,
            ).wait()

    issue_gathers(0, 0)

    def step(t, _):
        for b in range(2):
            g = t + b
            wait_gathers(g, b)
            @pl.when(g + 1 < NSTEP)
            def _():
                issue_gathers(g + 1, 1 - b)
            wb = pltpu.async_copy(
                rows_v.at[b],
                out_hbm.at[pl.ds(wid * B_PER_W + g * CHUNK, CHUNK)],
                sem_w,
            )
            wb.wait()
        return ()

    lax.fori_loop(0, NSTEP // 2, lambda t, c: step(t * 2, c), (),
                  unroll=False)


@jax.jit
def _gather(idx3, table):
    mesh = plsc.VectorSubcoreMesh(
        core_axis_name="c", subcore_axis_name="s", num_cores=NC,
        num_subcores=NS)
    return pl.kernel(
        _body,
        out_type=jax.ShapeDtypeStruct((B_TOTAL, EMBED_DIM), jnp.float32),
        mesh=mesh,
        scratch_types=[
            pltpu.VMEM((NSEG, SEG), jnp.int32),
            pltpu.VMEM((2, CHUNK, EMBED_DIM), jnp.float32),
            pltpu.SemaphoreType.DMA,
            pltpu.SemaphoreType.DMA,
        ],
        compiler_params=pltpu.CompilerParams(use_tc_tiling_on_sc=False),
    )(idx3, table)


def kernel(inputs, table):
    idx3 = inputs.reshape(NW, NSEG, SEG)
    return _gather(idx3, table)


# speedup vs baseline: 1.4063x; 1.4036x over previous
import jax
import jax.numpy as jnp
from jax import lax
from jax.experimental import pallas as pl
from jax.experimental.pallas import tpu as pltpu
from jax.experimental.pallas import tpu_sc as plsc

NUM_CATEGORIES = 1000000
EMBED_DIM = 64
BATCH = 4096
HIST_LEN = 50

NC = 2
NS = 16
NW = NC * NS

B_TOTAL = BATCH * HIST_LEN
B_PER_W = B_TOTAL // NW
ROWS = 64
NBATCH = B_PER_W // ROWS


def _body(idx_hbm, table_hbm, out_hbm, idx_v, rows_v, sem_g, sem_w):
    wid = lax.axis_index("s") * NC + lax.axis_index("c")
    base = wid * B_PER_W
    pltpu.sync_copy(idx_hbm.at[pl.ds(base, B_PER_W)], idx_v)

    def issue(g, b):
        for v16 in range(ROWS // 16):
            vec = idx_v[pl.ds(g * ROWS + v16 * 16, 16)]
            for j in range(16):
                pltpu.async_copy(
                    table_hbm.at[pl.ds(vec[j], 1)],
                    rows_v.at[b].at[pl.ds(v16 * 16 + j, 1)],
                    sem_g,
                )

    def drain(b):
        for j in range(ROWS):
            pltpu.make_async_copy(
                table_hbm.at[pl.ds(0, 1)],
                rows_v.at[b].at[pl.ds(j, 1)],
                sem_g,
            ).wait()

    issue(0, 0)

    def step(t, _):
        for p in range(2):
            g = 2 * t + p
            drain(p)
            @pl.when(g + 1 < NBATCH)
            def _():
                issue(g + 1, 1 - p)
            wb = pltpu.async_copy(
                rows_v.at[p],
                out_hbm.at[pl.ds(base + g * ROWS, ROWS)],
                sem_w,
            )
            wb.wait()
        return ()

    lax.fori_loop(0, NBATCH // 2, step, (), unroll=False)


@jax.jit
def _gather(idx1, table):
    mesh = plsc.VectorSubcoreMesh(
        core_axis_name="c", subcore_axis_name="s", num_cores=NC,
        num_subcores=NS)
    return pl.kernel(
        _body,
        out_type=jax.ShapeDtypeStruct((B_TOTAL, EMBED_DIM), jnp.float32),
        mesh=mesh,
        scratch_types=[
            pltpu.VMEM((B_PER_W,), jnp.int32),
            pltpu.VMEM((2, ROWS, EMBED_DIM), jnp.float32),
            pltpu.SemaphoreType.DMA,
            pltpu.SemaphoreType.DMA,
        ],
        compiler_params=pltpu.CompilerParams(use_tc_tiling_on_sc=True),
    )(idx1, table)


def kernel(inputs, table):
    return _gather(inputs.reshape(-1), table)


# speedup vs baseline: 1.5028x; 1.0686x over previous
import jax
import jax.numpy as jnp
from jax import lax
from jax.experimental import pallas as pl
from jax.experimental.pallas import tpu as pltpu
from jax.experimental.pallas import tpu_sc as plsc

NUM_CATEGORIES = 1000000
EMBED_DIM = 64
BATCH = 4096
HIST_LEN = 50

NC = 2
NS = 16
NW = NC * NS

B_TOTAL = BATCH * HIST_LEN
B_PER_W = B_TOTAL // NW
ROWS = 128
NBATCH = B_PER_W // ROWS


def _body(idx_hbm, table_hbm, out_hbm, idx_v, rows_v, sem_g, sem_w):
    wid = lax.axis_index("s") * NC + lax.axis_index("c")
    base = wid * B_PER_W
    pltpu.sync_copy(idx_hbm.at[pl.ds(base, B_PER_W)], idx_v)

    def issue(g, b):
        for v16 in range(ROWS // 16):
            vec = idx_v[pl.ds(g * ROWS + v16 * 16, 16)]
            offs = [vec[j] for j in range(16)]
            for j in range(16):
                pltpu.async_copy(
                    table_hbm.at[pl.ds(offs[j], 1)],
                    rows_v.at[b].at[pl.ds(v16 * 16 + j, 1)],
                    sem_g,
                )

    def drain(b):
        for w in range(ROWS // 16):
            pltpu.make_async_copy(
                table_hbm.at[pl.ds(0, 16)],
                rows_v.at[b].at[pl.ds(w * 16, 16)],
                sem_g,
            ).wait()

    issue(0, 0)

    def step(t, _):
        for p in range(2):
            g = 2 * t + p
            drain(p)
            @pl.when(g + 1 < NBATCH)
            def _():
                issue(g + 1, 1 - p)
            wb = pltpu.async_copy(
                rows_v.at[p],
                out_hbm.at[pl.ds(base + g * ROWS, ROWS)],
                sem_w,
            )
            wb.wait()
        return ()

    lax.fori_loop(0, NBATCH // 2, step, (), unroll=False)


@jax.jit
def _gather(idx1, table):
    mesh = plsc.VectorSubcoreMesh(
        core_axis_name="c", subcore_axis_name="s", num_cores=NC,
        num_subcores=NS)
    return pl.kernel(
        _body,
        out_type=jax.ShapeDtypeStruct((B_TOTAL, EMBED_DIM), jnp.float32),
        mesh=mesh,
        scratch_types=[
            pltpu.VMEM((B_PER_W,), jnp.int32),
            pltpu.VMEM((2, ROWS, EMBED_DIM), jnp.float32),
            pltpu.SemaphoreType.DMA,
            pltpu.SemaphoreType.DMA,
        ],
        compiler_params=pltpu.CompilerParams(use_tc_tiling_on_sc=True),
    )(idx1, table)


def kernel(inputs, table):
    return _gather(inputs.reshape(-1), table)


# speedup vs baseline: 1.5262x; 1.0155x over previous
import jax
import jax.numpy as jnp
from jax import lax
from jax.experimental import pallas as pl
from jax.experimental.pallas import tpu as pltpu
from jax.experimental.pallas import tpu_sc as plsc

NUM_CATEGORIES = 1000000
EMBED_DIM = 64
BATCH = 4096
HIST_LEN = 50

NC = 2
NS = 16
NW = NC * NS

B_TOTAL = BATCH * HIST_LEN
B_PER_W = B_TOTAL // NW
ROWS = 64
NBUF = 4
NBATCH = B_PER_W // ROWS
PREF = 2


def _body(idx_hbm, table_hbm, out_hbm, idx_v, rows_v,
          sem_g0, sem_g1, sem_w0, sem_w1):
    wid = lax.axis_index("s") * NC + lax.axis_index("c")
    base = wid * B_PER_W
    sems_g = (sem_g0, sem_g1)
    sems_w = (sem_w0, sem_w1)
    pltpu.sync_copy(idx_hbm.at[pl.ds(base, B_PER_W)], idx_v)

    def issue(g, b, par):
        for v16 in range(ROWS // 16):
            vec = idx_v[pl.ds(g * ROWS + v16 * 16, 16)]
            offs = [vec[j] for j in range(16)]
            for j in range(16):
                pltpu.async_copy(
                    table_hbm.at[pl.ds(offs[j], 1)],
                    rows_v.at[b].at[pl.ds(v16 * 16 + j, 1)],
                    sems_g[par],
                )

    def drain_gather(b, par):
        pltpu.make_async_copy(
            table_hbm.at[pl.ds(0, ROWS)],
            rows_v.at[b],
            sems_g[par],
        ).wait()

    def start_wb(g, b, par):
        pltpu.async_copy(
            rows_v.at[b],
            out_hbm.at[pl.ds(base + g * ROWS, ROWS)],
            sems_w[par],
        )

    def wait_wb(g, b, par):
        pltpu.make_async_copy(
            rows_v.at[b],
            out_hbm.at[pl.ds(base + g * ROWS, ROWS)],
            sems_w[par],
        ).wait()

    for g0 in range(PREF):
        issue(g0, g0, g0 % 2)

    def step(t, _):
        for p in range(NBUF):
            g = NBUF * t + p
            par = p % 2
            drain_gather(p, par)
            slot = (p + PREF) % NBUF
            @pl.when(g + PREF < NBATCH)
            def _():
                @pl.when(g + PREF - NBUF >= 0)
                def _():
                    wait_wb(g + PREF - NBUF, slot, slot % 2)
                issue(g + PREF, slot, slot % 2)
            start_wb(g, p, par)
        return ()

    lax.fori_loop(0, NBATCH // NBUF, step, (), unroll=False)

    for g in range(NBATCH - NBUF, NBATCH):
        wait_wb(g, g % NBUF, g % 2)


@jax.jit
def _gather(idx1, table):
    mesh = plsc.VectorSubcoreMesh(
        core_axis_name="c", subcore_axis_name="s", num_cores=NC,
        num_subcores=NS)
    return pl.kernel(
        _body,
        out_type=jax.ShapeDtypeStruct((B_TOTAL, EMBED_DIM), jnp.float32),
        mesh=mesh,
        scratch_types=[
            pltpu.VMEM((B_PER_W,), jnp.int32),
            pltpu.VMEM((NBUF, ROWS, EMBED_DIM), jnp.float32),
            pltpu.SemaphoreType.DMA,
            pltpu.SemaphoreType.DMA,
            pltpu.SemaphoreType.DMA,
            pltpu.SemaphoreType.DMA,
        ],
        compiler_params=pltpu.CompilerParams(use_tc_tiling_on_sc=True),
    )(idx1, table)


def kernel(inputs, table):
    return _gather(inputs.reshape(-1), table)


# speedup vs baseline: 1.5318x; 1.0037x over previous
import jax
import jax.numpy as jnp
from jax import lax
from jax.experimental import pallas as pl
from jax.experimental.pallas import tpu as pltpu
from jax.experimental.pallas import tpu_sc as plsc

NUM_CATEGORIES = 1000000
EMBED_DIM = 64
BATCH = 4096
HIST_LEN = 50

NC = 2
NS = 16
NW = NC * NS

B_TOTAL = BATCH * HIST_LEN
B_PER_W = B_TOTAL // NW
ROWS = 80
NBUF = 4
NBATCH = B_PER_W // ROWS
PREF = 2


def _body(idx_hbm, table_hbm, out_hbm, idx_v, rows_v,
          sem_g0, sem_g1, sem_w0, sem_w1):
    wid = lax.axis_index("s") * NC + lax.axis_index("c")
    base = wid * B_PER_W
    sems_g = (sem_g0, sem_g1)
    sems_w = (sem_w0, sem_w1)
    pltpu.sync_copy(idx_hbm.at[pl.ds(base, B_PER_W)], idx_v)

    def issue(g, b, par):
        for v16 in range(ROWS // 16):
            vec = idx_v[pl.ds(g * ROWS + v16 * 16, 16)]
            offs = [vec[j] for j in range(16)]
            for j in range(16):
                pltpu.async_copy(
                    table_hbm.at[pl.ds(offs[j], 1)],
                    rows_v.at[b].at[pl.ds(v16 * 16 + j, 1)],
                    sems_g[par],
                )

    def drain_gather(b, par):
        pltpu.make_async_copy(
            table_hbm.at[pl.ds(0, ROWS)],
            rows_v.at[b],
            sems_g[par],
        ).wait()

    def start_wb(g, b, par):
        pltpu.async_copy(
            rows_v.at[b],
            out_hbm.at[pl.ds(base + g * ROWS, ROWS)],
            sems_w[par],
        )

    def wait_wb(g, b, par):
        pltpu.make_async_copy(
            rows_v.at[b],
            out_hbm.at[pl.ds(base + g * ROWS, ROWS)],
            sems_w[par],
        ).wait()

    for g0 in range(PREF):
        issue(g0, g0, g0 % 2)

    def step(t, _):
        for p in range(NBUF):
            g = NBUF * t + p
            par = p % 2
            drain_gather(p, par)
            slot = (p + PREF) % NBUF
            @pl.when(g + PREF < NBATCH)
            def _():
                @pl.when(g + PREF - NBUF >= 0)
                def _():
                    wait_wb(g + PREF - NBUF, slot, slot % 2)
                issue(g + PREF, slot, slot % 2)
            start_wb(g, p, par)
        return ()

    lax.fori_loop(0, NBATCH // NBUF, step, (), unroll=False)

    for g in range(NBATCH - NBUF, NBATCH):
        wait_wb(g, g % NBUF, g % 2)


@jax.jit
def _gather(idx1, table):
    mesh = plsc.VectorSubcoreMesh(
        core_axis_name="c", subcore_axis_name="s", num_cores=NC,
        num_subcores=NS)
    return pl.kernel(
        _body,
        out_type=jax.ShapeDtypeStruct((B_TOTAL, EMBED_DIM), jnp.float32),
        mesh=mesh,
        scratch_types=[
            pltpu.VMEM((B_PER_W,), jnp.int32),
            pltpu.VMEM((NBUF, ROWS, EMBED_DIM), jnp.float32),
            pltpu.SemaphoreType.DMA,
            pltpu.SemaphoreType.DMA,
            pltpu.SemaphoreType.DMA,
            pltpu.SemaphoreType.DMA,
        ],
        compiler_params=pltpu.CompilerParams(use_tc_tiling_on_sc=True),
    )(idx1, table)


def kernel(inputs, table):
    return _gather(inputs.reshape(-1), table)


# speedup vs baseline: 1.5416x; 1.0064x over previous
import jax
import jax.numpy as jnp
from jax import lax
from jax.experimental import pallas as pl
from jax.experimental.pallas import tpu as pltpu
from jax.experimental.pallas import tpu_sc as plsc

NUM_CATEGORIES = 1000000
EMBED_DIM = 64
BATCH = 4096
HIST_LEN = 50

NC = 2
NS = 16
NW = NC * NS

B_TOTAL = BATCH * HIST_LEN
B_PER_W = B_TOTAL // NW
ROWS = 160
NBUF = 4
NBATCH = B_PER_W // ROWS
PREF = 2


def _body(idx_hbm, table_hbm, out_hbm, idx_v, rows_v,
          sem_g0, sem_g1, sem_w0, sem_w1):
    wid = lax.axis_index("s") * NC + lax.axis_index("c")
    base = wid * B_PER_W
    sems_g = (sem_g0, sem_g1)
    sems_w = (sem_w0, sem_w1)
    pltpu.sync_copy(idx_hbm.at[pl.ds(base, B_PER_W)], idx_v)

    def issue(g, b, par):
        for v16 in range(ROWS // 16):
            vec = idx_v[pl.ds(g * ROWS + v16 * 16, 16)]
            offs = [vec[j] for j in range(16)]
            for j in range(16):
                pltpu.async_copy(
                    table_hbm.at[pl.ds(offs[j], 1)],
                    rows_v.at[b].at[pl.ds(v16 * 16 + j, 1)],
                    sems_g[par],
                )

    def drain_gather(b, par):
        pltpu.make_async_copy(
            table_hbm.at[pl.ds(0, ROWS)],
            rows_v.at[b],
            sems_g[par],
        ).wait()

    def start_wb(g, b, par):
        pltpu.async_copy(
            rows_v.at[b],
            out_hbm.at[pl.ds(base + g * ROWS, ROWS)],
            sems_w[par],
        )

    def wait_wb(g, b, par):
        pltpu.make_async_copy(
            rows_v.at[b],
            out_hbm.at[pl.ds(base + g * ROWS, ROWS)],
            sems_w[par],
        ).wait()

    for g0 in range(PREF):
        issue(g0, g0, g0 % 2)

    def step(t, _):
        for p in range(NBUF):
            g = NBUF * t + p
            par = p % 2
            drain_gather(p, par)
            slot = (p + PREF) % NBUF
            @pl.when(g + PREF < NBATCH)
            def _():
                @pl.when(g + PREF - NBUF >= 0)
                def _():
                    wait_wb(g + PREF - NBUF, slot, slot % 2)
                issue(g + PREF, slot, slot % 2)
            start_wb(g, p, par)
        return ()

    lax.fori_loop(0, NBATCH // NBUF, step, (), unroll=False)

    for g in range(NBATCH - NBUF, NBATCH):
        wait_wb(g, g % NBUF, g % 2)


@jax.jit
def _gather(idx1, table):
    mesh = plsc.VectorSubcoreMesh(
        core_axis_name="c", subcore_axis_name="s", num_cores=NC,
        num_subcores=NS)
    return pl.kernel(
        _body,
        out_type=jax.ShapeDtypeStruct((B_TOTAL, EMBED_DIM), jnp.float32),
        mesh=mesh,
        scratch_types=[
            pltpu.VMEM((B_PER_W,), jnp.int32),
            pltpu.VMEM((NBUF, ROWS, EMBED_DIM), jnp.float32),
            pltpu.SemaphoreType.DMA,
            pltpu.SemaphoreType.DMA,
            pltpu.SemaphoreType.DMA,
            pltpu.SemaphoreType.DMA,
        ],
        compiler_params=pltpu.CompilerParams(use_tc_tiling_on_sc=True),
    )(idx1, table)


def kernel(inputs, table):
    return _gather(inputs.reshape(-1), table)
